# Initial kernel scaffold; baseline (speedup 1.0000x reference)
import jax
import jax.numpy as jnp
from jax.experimental import pallas as pl


def kernel(x, edge_index, W1, b1, W2, b2):
    raise NotImplementedError("write your pallas kernel here")


# speedup vs baseline: 67.4807x; 67.4807x over previous
import functools

import jax
import jax.numpy as jnp
from jax import lax
from jax.experimental import pallas as pl
from jax.experimental.pallas import tpu as pltpu
from jax.experimental.pallas import tpu_sc as plsc

N = 100000
E = 6400000
NC, NS = 2, 16
NW = NC * NS
CHUNK = 128
NCH = E // CHUNK
K = 8
NGRP = NCH // K
NPAD = 102400
RPS = N // NS
SLC = 6256
BB = 368
NB = SLC // BB
D1 = 16

_mesh = plsc.VectorSubcoreMesh(
    core_axis_name="c", subcore_axis_name="s", num_cores=NC, num_subcores=NS)
_sc_params = pltpu.CompilerParams(use_tc_tiling_on_sc=False)


def _row_off(s):
  q = s * RPS
  return pl.multiple_of(q - lax.rem(q, 8), 8)


def _bounds(c, s):
  w = s * NC + c
  return (NGRP * w) // NW, (NGRP * (w + 1)) // NW


@functools.partial(
    pl.kernel,
    out_type=jax.ShapeDtypeStruct((NC * NPAD,), jnp.float32),
    mesh=_mesh,
    compiler_params=_sc_params,
    scratch_types=[
        pltpu.VMEM((K, CHUNK), jnp.int32),
        pltpu.VMEM((CHUNK,), jnp.float32),
        pltpu.VMEM((BB,), jnp.float32),
        pltpu.VMEM_SHARED((N,), jnp.float32),
        pltpu.SemaphoreType.DMA,
    ],
)
def _k1_deg(dst3, z1, degp, didx, ones_v, zb, acc, sem):
  c = lax.axis_index("c")
  s = lax.axis_index("s")
  off = _row_off(s)
  pltpu.sync_copy(z1, zb)

  def zi(j, carry):
    o = pl.multiple_of(off + j * BB, 8)
    pltpu.sync_copy(zb, acc.at[pl.ds(o, BB)])
    return carry

  lax.fori_loop(0, NB, zi, 0)
  for i in range(CHUNK // 16):
    ones_v[pl.ds(i * 16, 16)] = jnp.ones((16,), jnp.float32)
  plsc.subcore_barrier()

  gstart, gend = _bounds(c, s)

  def grp(gi, carry):
    g0 = pl.multiple_of(gi * K, K)
    pltpu.sync_copy(dst3.at[pl.ds(g0, K), :], didx)
    descs = [
        pltpu.async_copy(ones_v, acc.at[didx.at[j]], sem, add=True)
        for j in range(K)
    ]
    for dsc in descs:
      dsc.wait()
    return carry

  lax.fori_loop(gstart, gend, grp, 0)
  plsc.subcore_barrier()

  def wb(j, carry):
    o = pl.multiple_of(off + j * BB, 8)
    pltpu.sync_copy(acc.at[pl.ds(o, BB)], zb)
    pltpu.sync_copy(zb, degp.at[pl.ds(c * NPAD + o, BB)])
    return carry

  lax.fori_loop(0, NB, wb, 0)


@functools.partial(
    pl.kernel,
    out_type=jax.ShapeDtypeStruct((NC, NPAD, D1), jnp.float32),
    mesh=_mesh,
    compiler_params=_sc_params,
    scratch_types=[
        pltpu.VMEM((K, CHUNK), jnp.int32),
        pltpu.VMEM((K, CHUNK), jnp.int32),
        pltpu.VMEM((K, CHUNK, D1), jnp.float32),
        pltpu.VMEM((BB, D1), jnp.float32),
        pltpu.VMEM_SHARED((N, D1), jnp.float32),
        pltpu.SemaphoreType.DMA,
        pltpu.SemaphoreType.DMA,
    ],
)
def _k3_agg1(src3, dst3, g1, z10, accp, sidx, didx, rows, zb, acc, semg, sems):
  c = lax.axis_index("c")
  s = lax.axis_index("s")
  off = _row_off(s)
  pltpu.sync_copy(z10, zb)

  def zi(j, carry):
    o = pl.multiple_of(off + j * BB, 8)
    pltpu.sync_copy(zb, acc.at[pl.ds(o, BB), :])
    return carry

  lax.fori_loop(0, NB, zi, 0)
  plsc.subcore_barrier()

  gstart, gend = _bounds(c, s)

  def grp(gi, carry):
    g0 = pl.multiple_of(gi * K, K)
    pltpu.sync_copy(src3.at[pl.ds(g0, K), :], sidx)
    pltpu.sync_copy(dst3.at[pl.ds(g0, K), :], didx)
    gd = [
        pltpu.async_copy(g1.at[sidx.at[j]], rows.at[j], semg)
        for j in range(K)
    ]
    for dsc in gd:
      dsc.wait()
    sd = [
        pltpu.async_copy(rows.at[j], acc.at[didx.at[j]], sems, add=True)
        for j in range(K)
    ]
    for dsc in sd:
      dsc.wait()
    return carry

  lax.fori_loop(gstart, gend, grp, 0)
  plsc.subcore_barrier()

  def wb(j, carry):
    o = pl.multiple_of(off + j * BB, 8)
    pltpu.sync_copy(acc.at[pl.ds(o, BB), :], zb)
    pltpu.sync_copy(zb, accp.at[c, pl.ds(o, BB), :])
    return carry

  lax.fori_loop(0, NB, wb, 0)


@functools.partial(
    pl.kernel,
    out_type=[jax.ShapeDtypeStruct((NC * NPAD,), jnp.float32),
              jax.ShapeDtypeStruct((NC * NPAD,), jnp.float32)],
    mesh=_mesh,
    compiler_params=_sc_params,
    scratch_types=[
        pltpu.VMEM((K, CHUNK), jnp.int32),
        pltpu.VMEM((K, CHUNK), jnp.int32),
        pltpu.VMEM((K, CHUNK), jnp.float32),
        pltpu.VMEM((K, CHUNK), jnp.float32),
        pltpu.VMEM((BB,), jnp.float32),
        pltpu.VMEM_SHARED((N,), jnp.float32),
        pltpu.VMEM_SHARED((N,), jnp.float32),
        pltpu.SemaphoreType.DMA,
        pltpu.SemaphoreType.DMA,
    ],
)
def _k5_agg2(src3, dst3, g2a, g2b, z1, oa, ob, sidx, didx, rowsa, rowsb, zb,
             acca, accb, semg, sems):
  c = lax.axis_index("c")
  s = lax.axis_index("s")
  off = _row_off(s)
  pltpu.sync_copy(z1, zb)

  def zi(j, carry):
    o = pl.multiple_of(off + j * BB, 8)
    pltpu.sync_copy(zb, acca.at[pl.ds(o, BB)])
    pltpu.sync_copy(zb, accb.at[pl.ds(o, BB)])
    return carry

  lax.fori_loop(0, NB, zi, 0)
  plsc.subcore_barrier()

  gstart, gend = _bounds(c, s)

  def grp(gi, carry):
    g0 = pl.multiple_of(gi * K, K)
    pltpu.sync_copy(src3.at[pl.ds(g0, K), :], sidx)
    pltpu.sync_copy(dst3.at[pl.ds(g0, K), :], didx)
    gd = [
        pltpu.async_copy(g2a.at[sidx.at[j]], rowsa.at[j], semg)
        for j in range(K)
    ] + [
        pltpu.async_copy(g2b.at[sidx.at[j]], rowsb.at[j], semg)
        for j in range(K)
    ]
    for dsc in gd:
      dsc.wait()
    sd = [
        pltpu.async_copy(rowsa.at[j], acca.at[didx.at[j]], sems, add=True)
        for j in range(K)
    ] + [
        pltpu.async_copy(rowsb.at[j], accb.at[didx.at[j]], sems, add=True)
        for j in range(K)
    ]
    for dsc in sd:
      dsc.wait()
    return carry

  lax.fori_loop(gstart, gend, grp, 0)
  plsc.subcore_barrier()

  def wb(j, carry):
    o = pl.multiple_of(off + j * BB, 8)
    pltpu.sync_copy(acca.at[pl.ds(o, BB)], zb)
    pltpu.sync_copy(zb, oa.at[pl.ds(c * NPAD + o, BB)])
    pltpu.sync_copy(accb.at[pl.ds(o, BB)], zb)
    pltpu.sync_copy(zb, ob.at[pl.ds(c * NPAD + o, BB)])
    return carry

  lax.fori_loop(0, NB, wb, 0)


_R = 1024
_G = NPAD // _R


def _k2_body(degp_ref, x_ref, g1_ref, d_ref):
  deg = degp_ref[0, :] + degp_ref[1, :] + 1.0
  dv = lax.rsqrt(deg)
  d_ref[...] = dv
  g1_ref[...] = jnp.pad(x_ref[...] * dv[:, None], ((0, 0), (0, D1 - 10)))


def _k2_scale(degp2, xp):
  return pl.pallas_call(
      _k2_body,
      grid=(_G,),
      in_specs=[
          pl.BlockSpec((2, _R), lambda i: (0, i)),
          pl.BlockSpec((_R, 10), lambda i: (i, 0)),
      ],
      out_specs=[
          pl.BlockSpec((_R, D1), lambda i: (i, 0)),
          pl.BlockSpec((_R,), lambda i: (i,)),
      ],
      out_shape=[
          jax.ShapeDtypeStruct((NPAD, D1), jnp.float32),
          jax.ShapeDtypeStruct((NPAD,), jnp.float32),
      ],
  )(degp2, xp)


def _k4_body(accp_ref, x_ref, d_ref, w1_ref, b1_ref, w2_ref,
             g2a_ref, g2b_ref, m2_ref):
  p = accp_ref[0][:, :10] + accp_ref[1][:, :10]
  dv = d_ref[...]
  aggx = dv[:, None] * p + (dv * dv)[:, None] * x_ref[...]
  h1 = jnp.dot(aggx, w1_ref[...], preferred_element_type=jnp.float32)
  h1 = jnp.maximum(h1 + b1_ref[...][None, :], 0.0)
  m2 = jnp.dot(h1, w2_ref[...], preferred_element_type=jnp.float32)
  m2_ref[...] = m2
  g2 = m2 * dv[:, None]
  g2a_ref[...] = g2[:, 0]
  g2b_ref[...] = g2[:, 1]


def _k4_dense(accp, xp, d, W1, b1, W2):
  return pl.pallas_call(
      _k4_body,
      grid=(_G,),
      in_specs=[
          pl.BlockSpec((2, _R, D1), lambda i: (0, i, 0)),
          pl.BlockSpec((_R, 10), lambda i: (i, 0)),
          pl.BlockSpec((_R,), lambda i: (i,)),
          pl.BlockSpec((10, 16), lambda i: (0, 0)),
          pl.BlockSpec((16,), lambda i: (0,)),
          pl.BlockSpec((16, 2), lambda i: (0, 0)),
      ],
      out_specs=[
          pl.BlockSpec((_R,), lambda i: (i,)),
          pl.BlockSpec((_R,), lambda i: (i,)),
          pl.BlockSpec((_R, 2), lambda i: (i, 0)),
      ],
      out_shape=[
          jax.ShapeDtypeStruct((NPAD,), jnp.float32),
          jax.ShapeDtypeStruct((NPAD,), jnp.float32),
          jax.ShapeDtypeStruct((NPAD, 2), jnp.float32),
      ],
  )(accp, xp, d, W1, b1, W2)


def _k6_body(qa_ref, qb_ref, d_ref, m2_ref, b2_ref, out_ref):
  qa = qa_ref[0, :] + qa_ref[1, :]
  qb = qb_ref[0, :] + qb_ref[1, :]
  dv = d_ref[...]
  q = jnp.stack([qa, qb], axis=1)
  out_ref[...] = (dv[:, None] * q + (dv * dv)[:, None] * m2_ref[...]
                  + b2_ref[...][None, :])


def _k6_out(qa, qb, d, m2, b2):
  return pl.pallas_call(
      _k6_body,
      grid=(_G,),
      in_specs=[
          pl.BlockSpec((2, _R), lambda i: (0, i)),
          pl.BlockSpec((2, _R), lambda i: (0, i)),
          pl.BlockSpec((_R,), lambda i: (i,)),
          pl.BlockSpec((_R, 2), lambda i: (i, 0)),
          pl.BlockSpec((2,), lambda i: (0,)),
      ],
      out_specs=pl.BlockSpec((_R, 2), lambda i: (i, 0)),
      out_shape=jax.ShapeDtypeStruct((NPAD, 2), jnp.float32),
  )(qa, qb, d, m2, b2)


def kernel(x, edge_index, W1, b1, W2, b2):
  ei = edge_index.astype(jnp.int32)
  src3 = ei[0].reshape(NCH, CHUNK)
  dst3 = ei[1].reshape(NCH, CHUNK)
  xp = jnp.pad(x, ((0, NPAD - N), (0, 0)))
  z1 = jnp.zeros((BB,), jnp.float32)
  z10 = jnp.zeros((BB, D1), jnp.float32)

  degp = _k1_deg(dst3, z1)
  g1, d = _k2_scale(degp.reshape(NC, NPAD), xp)
  accp = _k3_agg1(src3, dst3, g1, z10)
  g2a, g2b, m2 = _k4_dense(accp, xp, d, W1, b1, W2)
  qa, qb = _k5_agg2(src3, dst3, g2a, g2b, z1)
  out = _k6_out(qa.reshape(NC, NPAD), qb.reshape(NC, NPAD), d, m2, b2)
  return out[:N]


# speedup vs baseline: 74.6249x; 1.1059x over previous
import functools

import jax
import jax.numpy as jnp
from jax import lax
from jax.experimental import pallas as pl
from jax.experimental.pallas import tpu as pltpu
from jax.experimental.pallas import tpu_sc as plsc

N = 100000
E = 6400000
NC, NS = 2, 16
NW = NC * NS
CHUNK = 128
NCH = E // CHUNK
K = 8
NGRP = NCH // K
KE = 5
NGRPE = NCH // KE
NPAD = 102400
RPS = N // NS
SLC = 6256
BB = 368
NB = SLC // BB
D1 = 16

_mesh = plsc.VectorSubcoreMesh(
    core_axis_name="c", subcore_axis_name="s", num_cores=NC, num_subcores=NS)
_sc_params = pltpu.CompilerParams(use_tc_tiling_on_sc=False)


def _row_off(s):
  q = s * RPS
  return pl.multiple_of(q - lax.rem(q, 8), 8)


def _bounds(c, s, ngrp):
  w = s * NC + c
  return (ngrp * w) // NW, (ngrp * (w + 1)) // NW


@functools.partial(
    pl.kernel,
    out_type=jax.ShapeDtypeStruct((NC * NPAD,), jnp.float32),
    mesh=_mesh,
    compiler_params=_sc_params,
    scratch_types=[
        pltpu.VMEM((K, CHUNK), jnp.int32),
        pltpu.VMEM((CHUNK,), jnp.float32),
        pltpu.VMEM((BB,), jnp.float32),
        pltpu.VMEM_SHARED((N,), jnp.float32),
        pltpu.SemaphoreType.DMA,
    ],
)
def _k1_deg(dst3, z1, degp, didx, ones_v, zb, acc, sem):
  c = lax.axis_index("c")
  s = lax.axis_index("s")
  off = _row_off(s)
  pltpu.sync_copy(z1, zb)

  def zi(j, carry):
    o = pl.multiple_of(off + j * BB, 8)
    pltpu.sync_copy(zb, acc.at[pl.ds(o, BB)])
    return carry

  lax.fori_loop(0, NB, zi, 0)
  for i in range(CHUNK // 16):
    ones_v[pl.ds(i * 16, 16)] = jnp.ones((16,), jnp.float32)
  plsc.subcore_barrier()

  gstart, gend = _bounds(c, s, NGRP)

  def grp(gi, carry):
    g0 = pl.multiple_of(gi * K, K)
    pltpu.sync_copy(dst3.at[pl.ds(g0, K), :], didx)
    descs = [
        pltpu.async_copy(ones_v, acc.at[didx.at[j]], sem, add=True)
        for j in range(K)
    ]
    for dsc in descs:
      dsc.wait()
    return carry

  lax.fori_loop(gstart, gend, grp, 0)
  plsc.subcore_barrier()

  def wb(j, carry):
    o = pl.multiple_of(off + j * BB, 8)
    pltpu.sync_copy(acc.at[pl.ds(o, BB)], zb)
    pltpu.sync_copy(zb, degp.at[pl.ds(c * NPAD + o, BB)])
    return carry

  lax.fori_loop(0, NB, wb, 0)


@functools.partial(
    pl.kernel,
    out_type=jax.ShapeDtypeStruct((NC, NPAD, D1), jnp.float32),
    mesh=_mesh,
    compiler_params=_sc_params,
    scratch_types=[
        pltpu.VMEM((KE, CHUNK), jnp.int32),
        pltpu.VMEM((KE, CHUNK), jnp.int32),
        pltpu.VMEM((KE, CHUNK), jnp.int32),
        pltpu.VMEM((KE, CHUNK), jnp.int32),
        pltpu.VMEM((KE, CHUNK, D1), jnp.float32),
        pltpu.VMEM((KE, CHUNK, D1), jnp.float32),
        pltpu.VMEM((BB, D1), jnp.float32),
        pltpu.VMEM_SHARED((N, D1), jnp.float32),
        pltpu.SemaphoreType.DMA,
        pltpu.SemaphoreType.DMA,
        pltpu.SemaphoreType.DMA,
        pltpu.SemaphoreType.DMA,
    ],
)
def _edge_pass(src3, dst3, tab, z10, accp, sidxa, didxa, sidxb, didxb,
               rowsa, rowsb, zb, acc, semga, semsa, semgb, semsb):
  c = lax.axis_index("c")
  s = lax.axis_index("s")
  off = _row_off(s)
  pltpu.sync_copy(z10, zb)

  def zi(j, carry):
    o = pl.multiple_of(off + j * BB, 8)
    pltpu.sync_copy(zb, acc.at[pl.ds(o, BB), :])
    return carry

  lax.fori_loop(0, NB, zi, 0)
  plsc.subcore_barrier()

  gstart, gend = _bounds(c, s, NGRPE)
  n = gend - gstart

  def pair(t, carry):
    ga = (gstart + 2 * t) * KE
    gb = ga + KE
    pltpu.sync_copy(src3.at[pl.ds(ga, KE), :], sidxa)
    pltpu.sync_copy(dst3.at[pl.ds(ga, KE), :], didxa)
    gda = [
        pltpu.async_copy(tab.at[sidxa.at[j]], rowsa.at[j], semga)
        for j in range(KE)
    ]
    pltpu.sync_copy(src3.at[pl.ds(gb, KE), :], sidxb)
    pltpu.sync_copy(dst3.at[pl.ds(gb, KE), :], didxb)
    for dsc in gda:
      dsc.wait()
    sda = [
        pltpu.async_copy(rowsa.at[j], acc.at[didxa.at[j]], semsa, add=True)
        for j in range(KE)
    ]
    gdb = [
        pltpu.async_copy(tab.at[sidxb.at[j]], rowsb.at[j], semgb)
        for j in range(KE)
    ]
    for dsc in sda:
      dsc.wait()
    for dsc in gdb:
      dsc.wait()
    sdb = [
        pltpu.async_copy(rowsb.at[j], acc.at[didxb.at[j]], semsb, add=True)
        for j in range(KE)
    ]
    for dsc in sdb:
      dsc.wait()
    return carry

  lax.fori_loop(0, n // 2, pair, 0)

  @pl.when(lax.rem(n, 2) == 1)
  def _():
    g0 = (gend - 1) * KE
    pltpu.sync_copy(src3.at[pl.ds(g0, KE), :], sidxa)
    pltpu.sync_copy(dst3.at[pl.ds(g0, KE), :], didxa)
    gd = [
        pltpu.async_copy(tab.at[sidxa.at[j]], rowsa.at[j], semga)
        for j in range(KE)
    ]
    for dsc in gd:
      dsc.wait()
    sd = [
        pltpu.async_copy(rowsa.at[j], acc.at[didxa.at[j]], semsa, add=True)
        for j in range(KE)
    ]
    for dsc in sd:
      dsc.wait()

  plsc.subcore_barrier()

  def wb(j, carry):
    o = pl.multiple_of(off + j * BB, 8)
    pltpu.sync_copy(acc.at[pl.ds(o, BB), :], zb)
    pltpu.sync_copy(zb, accp.at[c, pl.ds(o, BB), :])
    return carry

  lax.fori_loop(0, NB, wb, 0)


_R = 1024
_G = NPAD // _R


def _k2_body(degp_ref, x_ref, g1_ref, d_ref):
  deg = degp_ref[0, :] + degp_ref[1, :] + 1.0
  dv = lax.rsqrt(deg)
  d_ref[...] = dv
  g1_ref[...] = jnp.pad(x_ref[...] * dv[:, None], ((0, 0), (0, D1 - 10)))


def _k2_scale(degp2, xp):
  return pl.pallas_call(
      _k2_body,
      grid=(_G,),
      in_specs=[
          pl.BlockSpec((2, _R), lambda i: (0, i)),
          pl.BlockSpec((_R, 10), lambda i: (i, 0)),
      ],
      out_specs=[
          pl.BlockSpec((_R, D1), lambda i: (i, 0)),
          pl.BlockSpec((_R,), lambda i: (i,)),
      ],
      out_shape=[
          jax.ShapeDtypeStruct((NPAD, D1), jnp.float32),
          jax.ShapeDtypeStruct((NPAD,), jnp.float32),
      ],
  )(degp2, xp)


def _k4_body(accp_ref, x_ref, d_ref, w1_ref, b1_ref, w2_ref,
             g2_ref, m2_ref):
  p = accp_ref[0][:, :10] + accp_ref[1][:, :10]
  dv = d_ref[...]
  aggx = dv[:, None] * p + (dv * dv)[:, None] * x_ref[...]
  h1 = jnp.dot(aggx, w1_ref[...], preferred_element_type=jnp.float32)
  h1 = jnp.maximum(h1 + b1_ref[...][None, :], 0.0)
  m2 = jnp.dot(h1, w2_ref[...], preferred_element_type=jnp.float32)
  m2_ref[...] = m2
  g2_ref[...] = jnp.pad(m2 * dv[:, None], ((0, 0), (0, D1 - 2)))


def _k4_dense(accp, xp, d, W1, b1, W2):
  return pl.pallas_call(
      _k4_body,
      grid=(_G,),
      in_specs=[
          pl.BlockSpec((2, _R, D1), lambda i: (0, i, 0)),
          pl.BlockSpec((_R, 10), lambda i: (i, 0)),
          pl.BlockSpec((_R,), lambda i: (i,)),
          pl.BlockSpec((10, 16), lambda i: (0, 0)),
          pl.BlockSpec((16,), lambda i: (0,)),
          pl.BlockSpec((16, 2), lambda i: (0, 0)),
      ],
      out_specs=[
          pl.BlockSpec((_R, D1), lambda i: (i, 0)),
          pl.BlockSpec((_R, 2), lambda i: (i, 0)),
      ],
      out_shape=[
          jax.ShapeDtypeStruct((NPAD, D1), jnp.float32),
          jax.ShapeDtypeStruct((NPAD, 2), jnp.float32),
      ],
  )(accp, xp, d, W1, b1, W2)


def _k6_body(accp2_ref, d_ref, m2_ref, b2_ref, out_ref):
  q = accp2_ref[0][:, :2] + accp2_ref[1][:, :2]
  dv = d_ref[...]
  out_ref[...] = (dv[:, None] * q + (dv * dv)[:, None] * m2_ref[...]
                  + b2_ref[...][None, :])


def _k6_out(accp2, d, m2, b2):
  return pl.pallas_call(
      _k6_body,
      grid=(_G,),
      in_specs=[
          pl.BlockSpec((2, _R, D1), lambda i: (0, i, 0)),
          pl.BlockSpec((_R,), lambda i: (i,)),
          pl.BlockSpec((_R, 2), lambda i: (i, 0)),
          pl.BlockSpec((2,), lambda i: (0,)),
      ],
      out_specs=pl.BlockSpec((_R, 2), lambda i: (i, 0)),
      out_shape=jax.ShapeDtypeStruct((NPAD, 2), jnp.float32),
  )(accp2, d, m2, b2)


def kernel(x, edge_index, W1, b1, W2, b2):
  ei = edge_index.astype(jnp.int32)
  src3 = ei[0].reshape(NCH, CHUNK)
  dst3 = ei[1].reshape(NCH, CHUNK)
  xp = jnp.pad(x, ((0, NPAD - N), (0, 0)))
  z1 = jnp.zeros((BB,), jnp.float32)
  z10 = jnp.zeros((BB, D1), jnp.float32)

  degp = _k1_deg(dst3, z1)
  g1, d = _k2_scale(degp.reshape(NC, NPAD), xp)
  accp = _edge_pass(src3, dst3, g1, z10)
  g2, m2 = _k4_dense(accp, xp, d, W1, b1, W2)
  accp2 = _edge_pass(src3, dst3, g2, z10)
  out = _k6_out(accp2, d, m2, b2)
  return out[:N]


# speedup vs baseline: 86.8520x; 1.1638x over previous
import functools

import jax
import jax.numpy as jnp
from jax import lax
from jax.experimental import pallas as pl
from jax.experimental.pallas import tpu as pltpu
from jax.experimental.pallas import tpu_sc as plsc

N = 100000
E = 6400000
NC, NS = 2, 16
NW = NC * NS
CHUNK = 128
NCH = E // CHUNK
K = 8
NGRP = NCH // K
KE = 5
NGRPE = NCH // KE
NPAD = 102400
RPS = N // NS
SLC = 6256
BB = 368
NB = SLC // BB
D1 = 16

_mesh = plsc.VectorSubcoreMesh(
    core_axis_name="c", subcore_axis_name="s", num_cores=NC, num_subcores=NS)
_sc_params = pltpu.CompilerParams(use_tc_tiling_on_sc=False)


def _row_off(s):
  q = s * RPS
  return pl.multiple_of(q - lax.rem(q, 8), 8)


def _bounds(c, s, ngrp):
  w = s * NC + c
  return (ngrp * w) // NW, (ngrp * (w + 1)) // NW


@functools.partial(
    pl.kernel,
    out_type=jax.ShapeDtypeStruct((NC * NPAD,), jnp.float32),
    mesh=_mesh,
    compiler_params=_sc_params,
    scratch_types=[
        pltpu.VMEM((K, CHUNK), jnp.int32),
        pltpu.VMEM((CHUNK,), jnp.float32),
        pltpu.VMEM((BB,), jnp.float32),
        pltpu.VMEM_SHARED((N,), jnp.float32),
        pltpu.SemaphoreType.DMA,
    ],
)
def _k1_deg(dst3, z1, degp, didx, ones_v, zb, acc, sem):
  c = lax.axis_index("c")
  s = lax.axis_index("s")
  off = _row_off(s)
  pltpu.sync_copy(z1, zb)

  def zi(j, carry):
    o = pl.multiple_of(off + j * BB, 8)
    pltpu.sync_copy(zb, acc.at[pl.ds(o, BB)])
    return carry

  lax.fori_loop(0, NB, zi, 0)
  for i in range(CHUNK // 16):
    ones_v[pl.ds(i * 16, 16)] = jnp.ones((16,), jnp.float32)
  plsc.subcore_barrier()

  gstart, gend = _bounds(c, s, NGRP)

  def grp(gi, carry):
    g0 = pl.multiple_of(gi * K, K)
    pltpu.sync_copy(dst3.at[pl.ds(g0, K), :], didx)
    descs = [
        pltpu.async_copy(ones_v, acc.at[didx.at[j]], sem, add=True)
        for j in range(K)
    ]
    for dsc in descs:
      dsc.wait()
    return carry

  lax.fori_loop(gstart, gend, grp, 0)
  plsc.subcore_barrier()

  def wb(j, carry):
    o = pl.multiple_of(off + j * BB, 8)
    pltpu.sync_copy(acc.at[pl.ds(o, BB)], zb)
    pltpu.sync_copy(zb, degp.at[pl.ds(c * NPAD + o, BB)])
    return carry

  lax.fori_loop(0, NB, wb, 0)


@functools.partial(
    pl.kernel,
    out_type=jax.ShapeDtypeStruct((NC, NPAD, D1), jnp.float32),
    mesh=_mesh,
    compiler_params=_sc_params,
    scratch_types=[
        pltpu.VMEM((KE, CHUNK), jnp.int32),
        pltpu.VMEM((KE, CHUNK), jnp.int32),
        pltpu.VMEM((KE, CHUNK), jnp.int32),
        pltpu.VMEM((KE, CHUNK), jnp.int32),
        pltpu.VMEM((KE, CHUNK, D1), jnp.float32),
        pltpu.VMEM((KE, CHUNK, D1), jnp.float32),
        pltpu.VMEM((BB, D1), jnp.float32),
        pltpu.VMEM_SHARED((N, D1), jnp.float32),
        pltpu.SemaphoreType.DMA,
        pltpu.SemaphoreType.DMA,
        pltpu.SemaphoreType.DMA,
        pltpu.SemaphoreType.DMA,
        pltpu.SemaphoreType.DMA,
        pltpu.SemaphoreType.DMA,
    ],
)
def _edge_pass(src3, dst3, tab, z10, accp, sidxa, didxa, sidxb, didxb,
               rowsa, rowsb, zb, acc, semga, semsa, semgb, semsb,
               semia, semib):
  c = lax.axis_index("c")
  s = lax.axis_index("s")
  off = _row_off(s)
  pltpu.sync_copy(z10, zb)

  def zi(j, carry):
    o = pl.multiple_of(off + j * BB, 8)
    pltpu.sync_copy(zb, acc.at[pl.ds(o, BB), :])
    return carry

  lax.fori_loop(0, NB, zi, 0)
  plsc.subcore_barrier()

  gstart, gend = _bounds(c, s, NGRPE)
  n = gend - gstart
  gmax = NCH - KE

  def _drain_idx(buf, sem):
    pltpu.make_async_copy(src3.at[pl.ds(0, KE), :], buf, sem).wait()

  ga0 = gstart * KE
  gb0 = jnp.minimum(ga0 + KE, gmax)
  pltpu.async_copy(src3.at[pl.ds(ga0, KE), :], sidxa, semia)
  pltpu.async_copy(dst3.at[pl.ds(ga0, KE), :], didxa, semia)
  pltpu.async_copy(src3.at[pl.ds(gb0, KE), :], sidxb, semib)
  pltpu.async_copy(dst3.at[pl.ds(gb0, KE), :], didxb, semib)

  def pair(t, carry):
    _drain_idx(sidxa, semia)
    _drain_idx(didxa, semia)
    gda = [
        pltpu.async_copy(tab.at[sidxa.at[j]], rowsa.at[j], semga)
        for j in range(KE)
    ]
    _drain_idx(sidxb, semib)
    _drain_idx(didxb, semib)
    for dsc in gda:
      dsc.wait()
    sda = [
        pltpu.async_copy(rowsa.at[j], acc.at[didxa.at[j]], semsa, add=True)
        for j in range(KE)
    ]
    gdb = [
        pltpu.async_copy(tab.at[sidxb.at[j]], rowsb.at[j], semgb)
        for j in range(KE)
    ]
    for dsc in sda:
      dsc.wait()
    ga_n = jnp.minimum((gstart + 2 * t + 2) * KE, gmax)
    pltpu.async_copy(src3.at[pl.ds(ga_n, KE), :], sidxa, semia)
    pltpu.async_copy(dst3.at[pl.ds(ga_n, KE), :], didxa, semia)
    for dsc in gdb:
      dsc.wait()
    sdb = [
        pltpu.async_copy(rowsb.at[j], acc.at[didxb.at[j]], semsb, add=True)
        for j in range(KE)
    ]
    for dsc in sdb:
      dsc.wait()
    gb_n = jnp.minimum((gstart + 2 * t + 3) * KE, gmax)
    pltpu.async_copy(src3.at[pl.ds(gb_n, KE), :], sidxb, semib)
    pltpu.async_copy(dst3.at[pl.ds(gb_n, KE), :], didxb, semib)
    return carry

  lax.fori_loop(0, n // 2, pair, 0)
  _drain_idx(sidxa, semia)
  _drain_idx(didxa, semia)
  _drain_idx(sidxb, semib)
  _drain_idx(didxb, semib)

  @pl.when(lax.rem(n, 2) == 1)
  def _():
    g0 = (gend - 1) * KE
    pltpu.sync_copy(src3.at[pl.ds(g0, KE), :], sidxa)
    pltpu.sync_copy(dst3.at[pl.ds(g0, KE), :], didxa)
    gd = [
        pltpu.async_copy(tab.at[sidxa.at[j]], rowsa.at[j], semga)
        for j in range(KE)
    ]
    for dsc in gd:
      dsc.wait()
    sd = [
        pltpu.async_copy(rowsa.at[j], acc.at[didxa.at[j]], semsa, add=True)
        for j in range(KE)
    ]
    for dsc in sd:
      dsc.wait()

  plsc.subcore_barrier()

  def wb(j, carry):
    o = pl.multiple_of(off + j * BB, 8)
    pltpu.sync_copy(acc.at[pl.ds(o, BB), :], zb)
    pltpu.sync_copy(zb, accp.at[c, pl.ds(o, BB), :])
    return carry

  lax.fori_loop(0, NB, wb, 0)


_R = 1024
_G = NPAD // _R


def _k2_body(degp_ref, x_ref, g1_ref, d_ref):
  deg = degp_ref[0, :] + degp_ref[1, :] + 1.0
  dv = lax.rsqrt(deg)
  d_ref[...] = dv
  g1_ref[...] = jnp.pad(x_ref[...] * dv[:, None], ((0, 0), (0, D1 - 10)))


def _k2_scale(degp2, xp):
  return pl.pallas_call(
      _k2_body,
      grid=(_G,),
      in_specs=[
          pl.BlockSpec((2, _R), lambda i: (0, i)),
          pl.BlockSpec((_R, 10), lambda i: (i, 0)),
      ],
      out_specs=[
          pl.BlockSpec((_R, D1), lambda i: (i, 0)),
          pl.BlockSpec((_R,), lambda i: (i,)),
      ],
      out_shape=[
          jax.ShapeDtypeStruct((NPAD, D1), jnp.float32),
          jax.ShapeDtypeStruct((NPAD,), jnp.float32),
      ],
  )(degp2, xp)


def _k4_body(accp_ref, x_ref, d_ref, w1_ref, b1_ref, w2_ref,
             g2_ref, m2_ref):
  p = accp_ref[0][:, :10] + accp_ref[1][:, :10]
  dv = d_ref[...]
  aggx = dv[:, None] * p + (dv * dv)[:, None] * x_ref[...]
  h1 = jnp.dot(aggx, w1_ref[...], preferred_element_type=jnp.float32)
  h1 = jnp.maximum(h1 + b1_ref[...][None, :], 0.0)
  m2 = jnp.dot(h1, w2_ref[...], preferred_element_type=jnp.float32)
  m2_ref[...] = m2
  g2_ref[...] = jnp.pad(m2 * dv[:, None], ((0, 0), (0, D1 - 2)))


def _k4_dense(accp, xp, d, W1, b1, W2):
  return pl.pallas_call(
      _k4_body,
      grid=(_G,),
      in_specs=[
          pl.BlockSpec((2, _R, D1), lambda i: (0, i, 0)),
          pl.BlockSpec((_R, 10), lambda i: (i, 0)),
          pl.BlockSpec((_R,), lambda i: (i,)),
          pl.BlockSpec((10, 16), lambda i: (0, 0)),
          pl.BlockSpec((16,), lambda i: (0,)),
          pl.BlockSpec((16, 2), lambda i: (0, 0)),
      ],
      out_specs=[
          pl.BlockSpec((_R, D1), lambda i: (i, 0)),
          pl.BlockSpec((_R, 2), lambda i: (i, 0)),
      ],
      out_shape=[
          jax.ShapeDtypeStruct((NPAD, D1), jnp.float32),
          jax.ShapeDtypeStruct((NPAD, 2), jnp.float32),
      ],
  )(accp, xp, d, W1, b1, W2)


def _k6_body(accp2_ref, d_ref, m2_ref, b2_ref, out_ref):
  q = accp2_ref[0][:, :2] + accp2_ref[1][:, :2]
  dv = d_ref[...]
  out_ref[...] = (dv[:, None] * q + (dv * dv)[:, None] * m2_ref[...]
                  + b2_ref[...][None, :])


def _k6_out(accp2, d, m2, b2):
  return pl.pallas_call(
      _k6_body,
      grid=(_G,),
      in_specs=[
          pl.BlockSpec((2, _R, D1), lambda i: (0, i, 0)),
          pl.BlockSpec((_R,), lambda i: (i,)),
          pl.BlockSpec((_R, 2), lambda i: (i, 0)),
          pl.BlockSpec((2,), lambda i: (0,)),
      ],
      out_specs=pl.BlockSpec((_R, 2), lambda i: (i, 0)),
      out_shape=jax.ShapeDtypeStruct((NPAD, 2), jnp.float32),
  )(accp2, d, m2, b2)


def kernel(x, edge_index, W1, b1, W2, b2):
  ei = edge_index.astype(jnp.int32)
  src3 = ei[0].reshape(NCH, CHUNK)
  dst3 = ei[1].reshape(NCH, CHUNK)
  xp = jnp.pad(x, ((0, NPAD - N), (0, 0)))
  z1 = jnp.zeros((BB,), jnp.float32)
  z10 = jnp.zeros((BB, D1), jnp.float32)

  degp = _k1_deg(dst3, z1)
  g1, d = _k2_scale(degp.reshape(NC, NPAD), xp)
  accp = _edge_pass(src3, dst3, g1, z10)
  g2, m2 = _k4_dense(accp, xp, d, W1, b1, W2)
  accp2 = _edge_pass(src3, dst3, g2, z10)
  out = _k6_out(accp2, d, m2, b2)
  return out[:N]


# speedup vs baseline: 100.6017x; 1.1583x over previous
import functools

import jax
import jax.numpy as jnp
from jax import lax
from jax.experimental import pallas as pl
from jax.experimental.pallas import tpu as pltpu
from jax.experimental.pallas import tpu_sc as plsc

N = 100000
E = 6400000
NC, NS = 2, 16
NW = NC * NS
CHUNK = 128
NCH = E // CHUNK
K = 8
NGRP = NCH // K
KE = 5
NGRPE = NCH // KE
NPAD = 102400
RPS = N // NS
SLC = 6256
BB = 368
NB = SLC // BB
D1 = 16

_mesh = plsc.VectorSubcoreMesh(
    core_axis_name="c", subcore_axis_name="s", num_cores=NC, num_subcores=NS)
_sc_params = pltpu.CompilerParams(use_tc_tiling_on_sc=False)


def _row_off(s):
  q = s * RPS
  return pl.multiple_of(q - lax.rem(q, 8), 8)


def _bounds(c, s, ngrp):
  w = s * NC + c
  return (ngrp * w) // NW, (ngrp * (w + 1)) // NW


@functools.partial(
    pl.kernel,
    out_type=jax.ShapeDtypeStruct((NC * NPAD,), jnp.float32),
    mesh=_mesh,
    compiler_params=_sc_params,
    scratch_types=[
        pltpu.VMEM((K, CHUNK), jnp.int32),
        pltpu.VMEM((CHUNK,), jnp.float32),
        pltpu.VMEM((BB,), jnp.float32),
        pltpu.VMEM_SHARED((N,), jnp.float32),
        pltpu.SemaphoreType.DMA,
    ],
)
def _k1_deg(dst3, z1, degp, didx, ones_v, zb, acc, sem):
  c = lax.axis_index("c")
  s = lax.axis_index("s")
  off = _row_off(s)
  pltpu.sync_copy(z1, zb)

  def zi(j, carry):
    o = pl.multiple_of(off + j * BB, 8)
    pltpu.sync_copy(zb, acc.at[pl.ds(o, BB)])
    return carry

  lax.fori_loop(0, NB, zi, 0)
  for i in range(CHUNK // 16):
    ones_v[pl.ds(i * 16, 16)] = jnp.ones((16,), jnp.float32)
  plsc.subcore_barrier()

  gstart, gend = _bounds(c, s, NGRP)

  def grp(gi, carry):
    g0 = pl.multiple_of(gi * K, K)
    pltpu.sync_copy(dst3.at[pl.ds(g0, K), :], didx)
    descs = [
        pltpu.async_copy(ones_v, acc.at[didx.at[j]], sem, add=True)
        for j in range(K)
    ]
    for dsc in descs:
      dsc.wait()
    return carry

  lax.fori_loop(gstart, gend, grp, 0)
  plsc.subcore_barrier()

  def wb(j, carry):
    o = pl.multiple_of(off + j * BB, 8)
    pltpu.sync_copy(acc.at[pl.ds(o, BB)], zb)
    pltpu.sync_copy(zb, degp.at[pl.ds(c * NPAD + o, BB)])
    return carry

  lax.fori_loop(0, NB, wb, 0)


def _make_edge_pass(W, spmem_table):
  scratch = [
      pltpu.VMEM((KE, CHUNK), jnp.int32),
      pltpu.VMEM((KE, CHUNK), jnp.int32),
      pltpu.VMEM((KE, CHUNK), jnp.int32),
      pltpu.VMEM((KE, CHUNK), jnp.int32),
      pltpu.VMEM((KE, CHUNK, W), jnp.float32),
      pltpu.VMEM((KE, CHUNK, W), jnp.float32),
      pltpu.VMEM((BB, W), jnp.float32),
      pltpu.VMEM_SHARED((N, W), jnp.float32),
      pltpu.SemaphoreType.DMA,
      pltpu.SemaphoreType.DMA,
      pltpu.SemaphoreType.DMA,
      pltpu.SemaphoreType.DMA,
      pltpu.SemaphoreType.DMA,
      pltpu.SemaphoreType.DMA,
  ]
  if spmem_table:
    scratch += [pltpu.VMEM((BB, W), jnp.float32),
                pltpu.VMEM_SHARED((N, W), jnp.float32)]

  @functools.partial(
      pl.kernel,
      out_type=jax.ShapeDtypeStruct((NC, NPAD, W), jnp.float32),
      mesh=_mesh,
      compiler_params=_sc_params,
      scratch_types=scratch,
  )
  def _edge_pass(src3, dst3, tabh, zW, accp, sidxa, didxa, sidxb, didxb,
                 rowsa, rowsb, zb, acc, semga, semsa, semgb, semsb,
                 semia, semib, *opt):
    c = lax.axis_index("c")
    s = lax.axis_index("s")
    off = _row_off(s)
    pltpu.sync_copy(zW, zb)
    if spmem_table:
      vt, tab = opt
    else:
      tab = tabh

    def zi(j, carry):
      o = pl.multiple_of(off + j * BB, 8)
      pltpu.sync_copy(zb, acc.at[pl.ds(o, BB), :])
      if spmem_table:
        pltpu.sync_copy(tabh.at[pl.ds(o, BB), :], vt)
        pltpu.sync_copy(vt, tab.at[pl.ds(o, BB), :])
      return carry

    lax.fori_loop(0, NB, zi, 0)
    plsc.subcore_barrier()

    gstart, gend = _bounds(c, s, NGRPE)
    n = gend - gstart
    gmax = NCH - KE

    def _drain_idx(buf, sem):
      pltpu.make_async_copy(src3.at[pl.ds(0, KE), :], buf, sem).wait()

    ga0 = gstart * KE
    gb0 = jnp.minimum(ga0 + KE, gmax)
    pltpu.async_copy(src3.at[pl.ds(ga0, KE), :], sidxa, semia)
    pltpu.async_copy(dst3.at[pl.ds(ga0, KE), :], didxa, semia)
    pltpu.async_copy(src3.at[pl.ds(gb0, KE), :], sidxb, semib)
    pltpu.async_copy(dst3.at[pl.ds(gb0, KE), :], didxb, semib)

    def pair(t, carry):
      _drain_idx(sidxa, semia)
      _drain_idx(didxa, semia)
      gda = [
          pltpu.async_copy(tab.at[sidxa.at[j]], rowsa.at[j], semga)
          for j in range(KE)
      ]
      _drain_idx(sidxb, semib)
      _drain_idx(didxb, semib)
      for dsc in gda:
        dsc.wait()
      sda = [
          pltpu.async_copy(rowsa.at[j], acc.at[didxa.at[j]], semsa, add=True)
          for j in range(KE)
      ]
      gdb = [
          pltpu.async_copy(tab.at[sidxb.at[j]], rowsb.at[j], semgb)
          for j in range(KE)
      ]
      for dsc in sda:
        dsc.wait()
      ga_n = jnp.minimum((gstart + 2 * t + 2) * KE, gmax)
      pltpu.async_copy(src3.at[pl.ds(ga_n, KE), :], sidxa, semia)
      pltpu.async_copy(dst3.at[pl.ds(ga_n, KE), :], didxa, semia)
      for dsc in gdb:
        dsc.wait()
      sdb = [
          pltpu.async_copy(rowsb.at[j], acc.at[didxb.at[j]], semsb, add=True)
          for j in range(KE)
      ]
      for dsc in sdb:
        dsc.wait()
      gb_n = jnp.minimum((gstart + 2 * t + 3) * KE, gmax)
      pltpu.async_copy(src3.at[pl.ds(gb_n, KE), :], sidxb, semib)
      pltpu.async_copy(dst3.at[pl.ds(gb_n, KE), :], didxb, semib)
      return carry

    lax.fori_loop(0, n // 2, pair, 0)
    _drain_idx(sidxa, semia)
    _drain_idx(didxa, semia)
    _drain_idx(sidxb, semib)
    _drain_idx(didxb, semib)

    @pl.when(lax.rem(n, 2) == 1)
    def _():
      g0 = (gend - 1) * KE
      pltpu.sync_copy(src3.at[pl.ds(g0, KE), :], sidxa)
      pltpu.sync_copy(dst3.at[pl.ds(g0, KE), :], didxa)
      gd = [
          pltpu.async_copy(tab.at[sidxa.at[j]], rowsa.at[j], semga)
          for j in range(KE)
      ]
      for dsc in gd:
        dsc.wait()
      sd = [
          pltpu.async_copy(rowsa.at[j], acc.at[didxa.at[j]], semsa, add=True)
          for j in range(KE)
      ]
      for dsc in sd:
        dsc.wait()

    plsc.subcore_barrier()

    def wb(j, carry):
      o = pl.multiple_of(off + j * BB, 8)
      pltpu.sync_copy(acc.at[pl.ds(o, BB), :], zb)
      pltpu.sync_copy(zb, accp.at[c, pl.ds(o, BB), :])
      return carry

    lax.fori_loop(0, NB, wb, 0)

  return _edge_pass


_edge16 = _make_edge_pass(D1, False)
_edge8 = _make_edge_pass(8, True)


_R = 1024
_G = NPAD // _R


def _k2_body(degp_ref, x_ref, g1_ref, d_ref):
  deg = degp_ref[0, :] + degp_ref[1, :] + 1.0
  dv = lax.rsqrt(deg)
  d_ref[...] = dv
  g1_ref[...] = jnp.pad(x_ref[...] * dv[:, None], ((0, 0), (0, D1 - 10)))


def _k2_scale(degp2, xp):
  return pl.pallas_call(
      _k2_body,
      grid=(_G,),
      in_specs=[
          pl.BlockSpec((2, _R), lambda i: (0, i)),
          pl.BlockSpec((_R, 10), lambda i: (i, 0)),
      ],
      out_specs=[
          pl.BlockSpec((_R, D1), lambda i: (i, 0)),
          pl.BlockSpec((_R,), lambda i: (i,)),
      ],
      out_shape=[
          jax.ShapeDtypeStruct((NPAD, D1), jnp.float32),
          jax.ShapeDtypeStruct((NPAD,), jnp.float32),
      ],
  )(degp2, xp)


def _k4_body(accp_ref, x_ref, d_ref, w1_ref, b1_ref, w2_ref,
             g2_ref, m2_ref):
  p = accp_ref[0][:, :10] + accp_ref[1][:, :10]
  dv = d_ref[...]
  aggx = dv[:, None] * p + (dv * dv)[:, None] * x_ref[...]
  h1 = jnp.dot(aggx, w1_ref[...], preferred_element_type=jnp.float32)
  h1 = jnp.maximum(h1 + b1_ref[...][None, :], 0.0)
  m2 = jnp.dot(h1, w2_ref[...], preferred_element_type=jnp.float32)
  m2_ref[...] = m2
  g2_ref[...] = jnp.pad(m2 * dv[:, None], ((0, 0), (0, 6)))


def _k4_dense(accp, xp, d, W1, b1, W2):
  return pl.pallas_call(
      _k4_body,
      grid=(_G,),
      in_specs=[
          pl.BlockSpec((2, _R, D1), lambda i: (0, i, 0)),
          pl.BlockSpec((_R, 10), lambda i: (i, 0)),
          pl.BlockSpec((_R,), lambda i: (i,)),
          pl.BlockSpec((10, 16), lambda i: (0, 0)),
          pl.BlockSpec((16,), lambda i: (0,)),
          pl.BlockSpec((16, 2), lambda i: (0, 0)),
      ],
      out_specs=[
          pl.BlockSpec((_R, 8), lambda i: (i, 0)),
          pl.BlockSpec((_R, 2), lambda i: (i, 0)),
      ],
      out_shape=[
          jax.ShapeDtypeStruct((NPAD, 8), jnp.float32),
          jax.ShapeDtypeStruct((NPAD, 2), jnp.float32),
      ],
  )(accp, xp, d, W1, b1, W2)


def _k6_body(accp2_ref, d_ref, m2_ref, b2_ref, out_ref):
  q = accp2_ref[0][:, :2] + accp2_ref[1][:, :2]
  dv = d_ref[...]
  out_ref[...] = (dv[:, None] * q + (dv * dv)[:, None] * m2_ref[...]
                  + b2_ref[...][None, :])


def _k6_out(accp2, d, m2, b2):
  return pl.pallas_call(
      _k6_body,
      grid=(_G,),
      in_specs=[
          pl.BlockSpec((2, _R, 8), lambda i: (0, i, 0)),
          pl.BlockSpec((_R,), lambda i: (i,)),
          pl.BlockSpec((_R, 2), lambda i: (i, 0)),
          pl.BlockSpec((2,), lambda i: (0,)),
      ],
      out_specs=pl.BlockSpec((_R, 2), lambda i: (i, 0)),
      out_shape=jax.ShapeDtypeStruct((NPAD, 2), jnp.float32),
  )(accp2, d, m2, b2)


def kernel(x, edge_index, W1, b1, W2, b2):
  ei = edge_index.astype(jnp.int32)
  src3 = ei[0].reshape(NCH, CHUNK)
  dst3 = ei[1].reshape(NCH, CHUNK)
  xp = jnp.pad(x, ((0, NPAD - N), (0, 0)))
  z1 = jnp.zeros((BB,), jnp.float32)
  z10 = jnp.zeros((BB, D1), jnp.float32)

  z8 = jnp.zeros((BB, 8), jnp.float32)
  degp = _k1_deg(dst3, z1)
  g1, d = _k2_scale(degp.reshape(NC, NPAD), xp)
  accp = _edge16(src3, dst3, g1, z10)
  g2, m2 = _k4_dense(accp, xp, d, W1, b1, W2)
  accp2 = _edge8(src3, dst3, g2, z8)
  out = _k6_out(accp2, d, m2, b2)
  return out[:N]


# speedup vs baseline: 106.2289x; 1.0559x over previous
import functools

import jax
import jax.numpy as jnp
from jax import lax
from jax.experimental import pallas as pl
from jax.experimental.pallas import tpu as pltpu
from jax.experimental.pallas import tpu_sc as plsc

N = 100000
E = 6400000
NC, NS = 2, 16
NW = NC * NS
CHUNK = 128
NCH = E // CHUNK
K = 8
NGRP = NCH // K
KE = 5
NGRPE = NCH // KE
NPAD = 102400
RPS = N // NS
SLC = 6256
BB = 368
NB = SLC // BB
D1 = 16

_mesh = plsc.VectorSubcoreMesh(
    core_axis_name="c", subcore_axis_name="s", num_cores=NC, num_subcores=NS)
_sc_params = pltpu.CompilerParams(use_tc_tiling_on_sc=False)


def _row_off(s):
  q = s * RPS
  return pl.multiple_of(q - lax.rem(q, 8), 8)


def _bounds(c, s, ngrp):
  w = s * NC + c
  return (ngrp * w) // NW, (ngrp * (w + 1)) // NW


@functools.partial(
    pl.kernel,
    out_type=jax.ShapeDtypeStruct((NC * NPAD,), jnp.float32),
    mesh=_mesh,
    compiler_params=_sc_params,
    scratch_types=[
        pltpu.VMEM((K, CHUNK), jnp.int32),
        pltpu.VMEM((K, CHUNK), jnp.int32),
        pltpu.VMEM((CHUNK,), jnp.float32),
        pltpu.VMEM((BB,), jnp.float32),
        pltpu.VMEM_SHARED((N,), jnp.float32),
        pltpu.SemaphoreType.DMA,
        pltpu.SemaphoreType.DMA,
        pltpu.SemaphoreType.DMA,
    ],
)
def _k1_deg(dst3, z1, degp, didxa, didxb, ones_v, zb, acc, sema, semsc,
            semib):
  c = lax.axis_index("c")
  s = lax.axis_index("s")
  off = _row_off(s)
  pltpu.sync_copy(z1, zb)

  def zi(j, carry):
    o = pl.multiple_of(off + j * BB, 8)
    pltpu.sync_copy(zb, acc.at[pl.ds(o, BB)])
    return carry

  lax.fori_loop(0, NB, zi, 0)
  for i in range(CHUNK // 16):
    ones_v[pl.ds(i * 16, 16)] = jnp.ones((16,), jnp.float32)
  plsc.subcore_barrier()

  gstart, gend = _bounds(c, s, NGRP)
  n = gend - gstart
  gmax1 = NCH - K

  def _drain1(buf, sem):
    pltpu.make_async_copy(dst3.at[pl.ds(0, K), :], buf, sem).wait()

  ga0 = gstart * K
  gb0 = jnp.minimum(ga0 + K, gmax1)
  pltpu.async_copy(dst3.at[pl.ds(ga0, K), :], didxa, sema)
  pltpu.async_copy(dst3.at[pl.ds(gb0, K), :], didxb, semib)

  def pair(t, carry):
    _drain1(didxa, sema)
    da = [
        pltpu.async_copy(ones_v, acc.at[didxa.at[j]], semsc, add=True)
        for j in range(K)
    ]
    for dsc in da:
      dsc.wait()
    ga_n = jnp.minimum((gstart + 2 * t + 2) * K, gmax1)
    pltpu.async_copy(dst3.at[pl.ds(ga_n, K), :], didxa, sema)
    _drain1(didxb, semib)
    db = [
        pltpu.async_copy(ones_v, acc.at[didxb.at[j]], semsc, add=True)
        for j in range(K)
    ]
    for dsc in db:
      dsc.wait()
    gb_n = jnp.minimum((gstart + 2 * t + 3) * K, gmax1)
    pltpu.async_copy(dst3.at[pl.ds(gb_n, K), :], didxb, semib)
    return carry

  lax.fori_loop(0, n // 2, pair, 0)
  _drain1(didxa, sema)
  _drain1(didxb, semib)

  @pl.when(lax.rem(n, 2) == 1)
  def _():
    g0 = (gend - 1) * K
    pltpu.sync_copy(dst3.at[pl.ds(g0, K), :], didxa)
    descs = [
        pltpu.async_copy(ones_v, acc.at[didxa.at[j]], semsc, add=True)
        for j in range(K)
    ]
    for dsc in descs:
      dsc.wait()

  plsc.subcore_barrier()

  def wb(j, carry):
    o = pl.multiple_of(off + j * BB, 8)
    pltpu.sync_copy(acc.at[pl.ds(o, BB)], zb)
    pltpu.sync_copy(zb, degp.at[pl.ds(c * NPAD + o, BB)])
    return carry

  lax.fori_loop(0, NB, wb, 0)


def _make_edge_pass(W, spmem_table):
  scratch = [
      pltpu.VMEM((KE, CHUNK), jnp.int32),
      pltpu.VMEM((KE, CHUNK), jnp.int32),
      pltpu.VMEM((KE, CHUNK), jnp.int32),
      pltpu.VMEM((KE, CHUNK), jnp.int32),
      pltpu.VMEM((KE, CHUNK, W), jnp.float32),
      pltpu.VMEM((KE, CHUNK, W), jnp.float32),
      pltpu.VMEM((BB, W), jnp.float32),
      pltpu.VMEM_SHARED((N, W), jnp.float32),
      pltpu.SemaphoreType.DMA,
      pltpu.SemaphoreType.DMA,
      pltpu.SemaphoreType.DMA,
      pltpu.SemaphoreType.DMA,
      pltpu.SemaphoreType.DMA,
      pltpu.SemaphoreType.DMA,
  ]
  if spmem_table:
    scratch += [pltpu.VMEM((BB, W), jnp.float32),
                pltpu.VMEM_SHARED((N, W), jnp.float32)]

  @functools.partial(
      pl.kernel,
      out_type=jax.ShapeDtypeStruct((NC, NPAD, W), jnp.float32),
      mesh=_mesh,
      compiler_params=_sc_params,
      scratch_types=scratch,
  )
  def _edge_pass(src3, dst3, tabh, zW, accp, sidxa, didxa, sidxb, didxb,
                 rowsa, rowsb, zb, acc, semga, semsa, semgb, semsb,
                 semia, semib, *opt):
    c = lax.axis_index("c")
    s = lax.axis_index("s")
    off = _row_off(s)
    pltpu.sync_copy(zW, zb)
    if spmem_table:
      vt, tab = opt
    else:
      tab = tabh

    def zi(j, carry):
      o = pl.multiple_of(off + j * BB, 8)
      pltpu.sync_copy(zb, acc.at[pl.ds(o, BB), :])
      if spmem_table:
        pltpu.sync_copy(tabh.at[pl.ds(o, BB), :], vt)
        pltpu.sync_copy(vt, tab.at[pl.ds(o, BB), :])
      return carry

    lax.fori_loop(0, NB, zi, 0)
    plsc.subcore_barrier()

    gstart, gend = _bounds(c, s, NGRPE)
    n = gend - gstart
    gmax = NCH - KE

    def _drain_idx(buf, sem):
      pltpu.make_async_copy(src3.at[pl.ds(0, KE), :], buf, sem).wait()

    ga0 = gstart * KE
    gb0 = jnp.minimum(ga0 + KE, gmax)
    pltpu.async_copy(src3.at[pl.ds(ga0, KE), :], sidxa, semia)
    pltpu.async_copy(dst3.at[pl.ds(ga0, KE), :], didxa, semia)
    pltpu.async_copy(src3.at[pl.ds(gb0, KE), :], sidxb, semib)
    pltpu.async_copy(dst3.at[pl.ds(gb0, KE), :], didxb, semib)

    def pair(t, carry):
      _drain_idx(sidxa, semia)
      _drain_idx(didxa, semia)
      gda = [
          pltpu.async_copy(tab.at[sidxa.at[j]], rowsa.at[j], semga)
          for j in range(KE)
      ]
      _drain_idx(sidxb, semib)
      _drain_idx(didxb, semib)
      for dsc in gda:
        dsc.wait()
      sda = [
          pltpu.async_copy(rowsa.at[j], acc.at[didxa.at[j]], semsa, add=True)
          for j in range(KE)
      ]
      gdb = [
          pltpu.async_copy(tab.at[sidxb.at[j]], rowsb.at[j], semgb)
          for j in range(KE)
      ]
      for dsc in sda:
        dsc.wait()
      ga_n = jnp.minimum((gstart + 2 * t + 2) * KE, gmax)
      pltpu.async_copy(src3.at[pl.ds(ga_n, KE), :], sidxa, semia)
      pltpu.async_copy(dst3.at[pl.ds(ga_n, KE), :], didxa, semia)
      for dsc in gdb:
        dsc.wait()
      sdb = [
          pltpu.async_copy(rowsb.at[j], acc.at[didxb.at[j]], semsb, add=True)
          for j in range(KE)
      ]
      for dsc in sdb:
        dsc.wait()
      gb_n = jnp.minimum((gstart + 2 * t + 3) * KE, gmax)
      pltpu.async_copy(src3.at[pl.ds(gb_n, KE), :], sidxb, semib)
      pltpu.async_copy(dst3.at[pl.ds(gb_n, KE), :], didxb, semib)
      return carry

    lax.fori_loop(0, n // 2, pair, 0)
    _drain_idx(sidxa, semia)
    _drain_idx(didxa, semia)
    _drain_idx(sidxb, semib)
    _drain_idx(didxb, semib)

    @pl.when(lax.rem(n, 2) == 1)
    def _():
      g0 = (gend - 1) * KE
      pltpu.sync_copy(src3.at[pl.ds(g0, KE), :], sidxa)
      pltpu.sync_copy(dst3.at[pl.ds(g0, KE), :], didxa)
      gd = [
          pltpu.async_copy(tab.at[sidxa.at[j]], rowsa.at[j], semga)
          for j in range(KE)
      ]
      for dsc in gd:
        dsc.wait()
      sd = [
          pltpu.async_copy(rowsa.at[j], acc.at[didxa.at[j]], semsa, add=True)
          for j in range(KE)
      ]
      for dsc in sd:
        dsc.wait()

    plsc.subcore_barrier()

    def wb(j, carry):
      o = pl.multiple_of(off + j * BB, 8)
      pltpu.sync_copy(acc.at[pl.ds(o, BB), :], zb)
      pltpu.sync_copy(zb, accp.at[c, pl.ds(o, BB), :])
      return carry

    lax.fori_loop(0, NB, wb, 0)

  return _edge_pass


_edge16 = _make_edge_pass(D1, False)
_edge8 = _make_edge_pass(8, True)


_R = 1024
_G = NPAD // _R


def _k2_body(degp_ref, x_ref, g1_ref, d_ref):
  deg = degp_ref[0, :] + degp_ref[1, :] + 1.0
  dv = lax.rsqrt(deg)
  d_ref[...] = dv
  g1_ref[...] = jnp.pad(x_ref[...] * dv[:, None], ((0, 0), (0, D1 - 10)))


def _k2_scale(degp2, xp):
  return pl.pallas_call(
      _k2_body,
      grid=(_G,),
      in_specs=[
          pl.BlockSpec((2, _R), lambda i: (0, i)),
          pl.BlockSpec((_R, 10), lambda i: (i, 0)),
      ],
      out_specs=[
          pl.BlockSpec((_R, D1), lambda i: (i, 0)),
          pl.BlockSpec((_R,), lambda i: (i,)),
      ],
      out_shape=[
          jax.ShapeDtypeStruct((NPAD, D1), jnp.float32),
          jax.ShapeDtypeStruct((NPAD,), jnp.float32),
      ],
  )(degp2, xp)


def _k4_body(accp_ref, x_ref, d_ref, w1_ref, b1_ref, w2_ref,
             g2_ref, m2_ref):
  p = accp_ref[0][:, :10] + accp_ref[1][:, :10]
  dv = d_ref[...]
  aggx = dv[:, None] * p + (dv * dv)[:, None] * x_ref[...]
  h1 = jnp.dot(aggx, w1_ref[...], preferred_element_type=jnp.float32)
  h1 = jnp.maximum(h1 + b1_ref[...][None, :], 0.0)
  m2 = jnp.dot(h1, w2_ref[...], preferred_element_type=jnp.float32)
  m2_ref[...] = m2
  g2_ref[...] = jnp.pad(m2 * dv[:, None], ((0, 0), (0, 6)))


def _k4_dense(accp, xp, d, W1, b1, W2):
  return pl.pallas_call(
      _k4_body,
      grid=(_G,),
      in_specs=[
          pl.BlockSpec((2, _R, D1), lambda i: (0, i, 0)),
          pl.BlockSpec((_R, 10), lambda i: (i, 0)),
          pl.BlockSpec((_R,), lambda i: (i,)),
          pl.BlockSpec((10, 16), lambda i: (0, 0)),
          pl.BlockSpec((16,), lambda i: (0,)),
          pl.BlockSpec((16, 2), lambda i: (0, 0)),
      ],
      out_specs=[
          pl.BlockSpec((_R, 8), lambda i: (i, 0)),
          pl.BlockSpec((_R, 2), lambda i: (i, 0)),
      ],
      out_shape=[
          jax.ShapeDtypeStruct((NPAD, 8), jnp.float32),
          jax.ShapeDtypeStruct((NPAD, 2), jnp.float32),
      ],
  )(accp, xp, d, W1, b1, W2)


def _k6_body(accp2_ref, d_ref, m2_ref, b2_ref, out_ref):
  q = accp2_ref[0][:, :2] + accp2_ref[1][:, :2]
  dv = d_ref[...]
  out_ref[...] = (dv[:, None] * q + (dv * dv)[:, None] * m2_ref[...]
                  + b2_ref[...][None, :])


def _k6_out(accp2, d, m2, b2):
  return pl.pallas_call(
      _k6_body,
      grid=(_G,),
      in_specs=[
          pl.BlockSpec((2, _R, 8), lambda i: (0, i, 0)),
          pl.BlockSpec((_R,), lambda i: (i,)),
          pl.BlockSpec((_R, 2), lambda i: (i, 0)),
          pl.BlockSpec((2,), lambda i: (0,)),
      ],
      out_specs=pl.BlockSpec((_R, 2), lambda i: (i, 0)),
      out_shape=jax.ShapeDtypeStruct((NPAD, 2), jnp.float32),
  )(accp2, d, m2, b2)


def kernel(x, edge_index, W1, b1, W2, b2):
  ei = edge_index.astype(jnp.int32)
  src3 = ei[0].reshape(NCH, CHUNK)
  dst3 = ei[1].reshape(NCH, CHUNK)
  xp = jnp.pad(x, ((0, NPAD - N), (0, 0)))
  z1 = jnp.zeros((BB,), jnp.float32)
  z10 = jnp.zeros((BB, D1), jnp.float32)

  z8 = jnp.zeros((BB, 8), jnp.float32)
  degp = _k1_deg(dst3, z1)
  g1, d = _k2_scale(degp.reshape(NC, NPAD), xp)
  accp = _edge16(src3, dst3, g1, z10)
  g2, m2 = _k4_dense(accp, xp, d, W1, b1, W2)
  accp2 = _edge8(src3, dst3, g2, z8)
  out = _k6_out(accp2, d, m2, b2)
  return out[:N]
